# Initial kernel scaffold; baseline (speedup 1.0000x reference)
import jax
import jax.numpy as jnp
from jax.experimental import pallas as pl


def kernel(x_a, x_b, edge_index_ab, edge_index_ba, W_ab, W_ba):
    raise NotImplementedError("write your pallas kernel here")


# speedup vs baseline: 4.6692x; 4.6692x over previous
import functools

import jax
import jax.numpy as jnp
from jax import lax
from jax.experimental import pallas as pl
from jax.experimental.pallas import tpu as pltpu
from jax.experimental.pallas import tpu_sc as plsc

N = 10000
E = 320000
D = 128
CH = 128
NTILES = 16
MAIN = 160
NCHUNK = MAIN * NTILES
EPAD = NCHUNK * CH
NPADROWS = 112
BCH = 16
NBLK = MAIN // BCH
ROWS_T = (N + NPADROWS) // NTILES
NACC = ROWS_T * NTILES
LAST = N - ROWS_T * (NTILES - 1)
CW = 16


def _sc_body(x_a, x_b, src_ab, dst_ab, src_ba, dst_ba, zfeat, zcnt, omsg,
             sums_o, cnts_o,
             acc, cacc, isrc, idst, rows, ones_v, sem):
    c = lax.axis_index("c")
    tid = lax.axis_index("s")

    def run_rel(rel, src_r, dst_r, x_r):
        base = tid * ROWS_T
        pltpu.sync_copy(zfeat, rows)
        pltpu.sync_copy(zcnt, ones_v)
        for off in (0, 128, 256, 384, 504):
            pltpu.sync_copy(rows, acc.at[pl.ds(base + off, CH)])
            pltpu.sync_copy(ones_v, cacc.at[pl.ds(base + off, CH)])
        pltpu.sync_copy(omsg, ones_v)
        plsc.subcore_barrier()

        def block(b, carry):
            base = pl.ds(tid * MAIN + b * BCH, BCH)
            pltpu.sync_copy(src_r.at[base], isrc)
            pltpu.sync_copy(dst_r.at[base], idst)

            def chunk(j, carry2):
                pltpu.async_copy(x_r.at[isrc.at[j]], rows, sem).wait()
                pltpu.sync_copy(rows, acc.at[idst.at[j]], add=True)
                pltpu.sync_copy(ones_v, cacc.at[idst.at[j]], add=True)
                return carry2

            lax.fori_loop(0, BCH, chunk, 0)
            return carry

        lax.fori_loop(0, NBLK, block, 0)
        plsc.subcore_barrier()

        def emit(off):
            sl = pl.ds(base + off, CH)
            pltpu.sync_copy(acc.at[sl], rows)
            pltpu.sync_copy(rows, sums_o.at[rel, sl])
            pltpu.sync_copy(cacc.at[sl], ones_v)
            pltpu.sync_copy(ones_v, cnts_o.at[rel, sl])

        @pl.when(tid < NTILES - 1)
        def _():
            for off in (0, 128, 256, 384, 504):
                emit(off)

        @pl.when(tid == NTILES - 1)
        def _():
            for off in (0, 128, 256, 384, LAST - CH):
                emit(off)

    @pl.when(c == 0)
    def _():
        run_rel(0, src_ba, dst_ba, x_b)

    @pl.when(c == 1)
    def _():
        run_rel(1, src_ab, dst_ab, x_a)


@functools.partial(
    pl.kernel,
    mesh=plsc.VectorSubcoreMesh(core_axis_name="c", subcore_axis_name="s"),
    out_type=[
        jax.ShapeDtypeStruct((2, N, D), jnp.float32),
        jax.ShapeDtypeStruct((2, N, CW), jnp.float32),
    ],
    scratch_types=[
        pltpu.VMEM_SHARED((NACC, D), jnp.float32),
        pltpu.VMEM_SHARED((NACC, CW), jnp.float32),
        pltpu.VMEM((BCH, CH), jnp.int32),
        pltpu.VMEM((BCH, CH), jnp.int32),
        pltpu.VMEM((CH, D), jnp.float32),
        pltpu.VMEM((CH, CW), jnp.float32),
        pltpu.SemaphoreType.DMA,
    ],
    compiler_params=pltpu.CompilerParams(use_tc_tiling_on_sc=False),
)
def _sc_aggregate(*refs):
    _sc_body(*refs)


def _tc_body(sums_ref, cnts_ref, w_ref, out_ref):
    s = sums_ref[0]
    cnt = jnp.maximum(cnts_ref[0][:, 0:1], 1.0)
    out_ref[0] = jnp.dot(s / cnt, w_ref[0], preferred_element_type=jnp.float32)


def _tc_finalize(sums, cnts, w_stack):
    blk = 1000
    return pl.pallas_call(
        _tc_body,
        grid=(2, N // blk),
        in_specs=[
            pl.BlockSpec((1, blk, D), lambda r, i: (r, i, 0)),
            pl.BlockSpec((1, blk, CW), lambda r, i: (r, i, 0)),
            pl.BlockSpec((1, D, D), lambda r, i: (r, 0, 0)),
        ],
        out_specs=pl.BlockSpec((1, blk, D), lambda r, i: (r, i, 0)),
        out_shape=jax.ShapeDtypeStruct((2, N, D), jnp.float32),
    )(sums, cnts, w_stack)


def _pad_edges(edge_index):
    npad = EPAD - E
    src = jnp.concatenate(
        [edge_index[0], jnp.zeros((npad,), jnp.int32)]).reshape(NCHUNK, CH)
    dst = jnp.concatenate(
        [edge_index[1],
         N + (jnp.arange(npad, dtype=jnp.int32) % NPADROWS)]).reshape(NCHUNK, CH)
    return src, dst


def kernel(x_a, x_b, edge_index_ab, edge_index_ba, W_ab, W_ba):
    src_ab, dst_ab = _pad_edges(edge_index_ab)
    src_ba, dst_ba = _pad_edges(edge_index_ba)
    zfeat = jnp.zeros((CH, D), jnp.float32)
    zcnt = jnp.zeros((CH, CW), jnp.float32)
    omsg = jnp.ones((CH, CW), jnp.float32)
    sums, cnts = _sc_aggregate(x_a, x_b, src_ab, dst_ab, src_ba, dst_ba,
                               zfeat, zcnt, omsg)
    w_stack = jnp.stack([W_ba, W_ab], axis=0)
    return _tc_finalize(sums, cnts, w_stack)
